# baseline (speedup 1.0000x reference)
import jax
import jax.numpy as jnp
from jax import lax
from jax.experimental import pallas as pl


_B, _H, _W, _C = 32, 512, 512, 3
_LANES = _W * _C
_ROWS = _H
_ROW_BLK = 256


def _rev3_kernel(x_ref, o_ref):
    x = x_ref[...]
    up2 = jnp.concatenate([x[:, 2:], x[:, :2]], axis=1)
    dn2 = jnp.concatenate([x[:, -2:], x[:, :-2]], axis=1)
    mod = lax.broadcasted_iota(jnp.int32, x.shape, 1) % 3
    o_ref[...] = jnp.where(mod == 0, up2, jnp.where(mod == 1, x, dn2))


def kernel(inputs):
    x = inputs.reshape(_B, _ROWS, _LANES)
    out = pl.pallas_call(
        _rev3_kernel,
        grid=(_B, _ROWS // _ROW_BLK),
        in_specs=[pl.BlockSpec((None, _ROW_BLK, _LANES), lambda b, r: (b, r, 0))],
        out_specs=pl.BlockSpec((None, _ROW_BLK, _LANES), lambda b, r: (b, r, 0)),
        out_shape=jax.ShapeDtypeStruct((_B, _ROWS, _LANES), jnp.float32),
    )(x)
    return out.reshape(_B, _H, _W, _C)
